# baseline (speedup 1.0000x reference)
import functools

import jax
import jax.numpy as jnp
from jax import lax
from jax.experimental import pallas as pl
from jax.experimental.pallas import tpu as pltpu
from jax.experimental.pallas import tpu_sc as plsc

VOCAB = 100000
DIM = 64
PAD_DIM = 128
BATCH = 4096
SEQ = 50
NUM_WORKERS = 32
BATCH_PER_W = BATCH // NUM_WORKERS
GROUP = 4
NBUF = 4
NUM_GROUPS = BATCH_PER_W // GROUP


def _make_kernel():
    mesh = plsc.VectorSubcoreMesh(core_axis_name="c", subcore_axis_name="s")

    @functools.partial(
        pl.kernel,
        mesh=mesh,
        out_type=jax.ShapeDtypeStruct((BATCH, SEQ, PAD_DIM), jnp.float32),
        scratch_types=[
            pltpu.VMEM((BATCH_PER_W, SEQ), jnp.int32),
            pltpu.VMEM((NBUF, GROUP, SEQ, PAD_DIM), jnp.float32),
            pltpu.SemaphoreType.DMA,
            pltpu.SemaphoreType.DMA,
        ],
        compiler_params=pltpu.CompilerParams(use_tc_tiling_on_sc=True),
    )
    def k(idx_hbm, table_hbm, out_hbm, idx_v, rows_v, sem_g, sem_s):
        wid = lax.axis_index("s") * 2 + lax.axis_index("c")
        base = wid * BATCH_PER_W

        pltpu.sync_copy(idx_hbm.at[pl.ds(base, BATCH_PER_W)], idx_v)

        def gather_descs(g):
            return [
                pltpu.make_async_copy(
                    table_hbm.at[idx_v.at[g * GROUP + j]],
                    rows_v.at[g % NBUF, j],
                    sem_g,
                )
                for j in range(GROUP)
            ]

        def store_desc(g):
            return pltpu.make_async_copy(
                rows_v.at[g % NBUF],
                out_hbm.at[pl.ds(base + g * GROUP, GROUP)],
                sem_s,
            )

        for g0 in range(2):
            for d in gather_descs(g0):
                d.start()

        def group_wait(g):
            pltpu.make_async_copy(
                out_hbm.at[pl.ds(0, GROUP)],
                rows_v.at[g % NBUF],
                sem_g,
            ).wait()

        def body(g, carry):
            group_wait(g)

            @pl.when(g >= 2)
            def _():
                store_desc(g - 2).wait()

            @pl.when(g < NUM_GROUPS - 2)
            def _():
                for d in gather_descs(g + 2):
                    d.start()

            store_desc(g).start()
            return carry

        lax.fori_loop(0, NUM_GROUPS, body, 0)
        store_desc(NUM_GROUPS - 2).wait()
        store_desc(NUM_GROUPS - 1).wait()

    return k


_gather_kernel = _make_kernel()


def kernel(indices, input_embeds):
    table_padded = jnp.zeros((VOCAB, PAD_DIM), jnp.float32).at[:, :DIM].set(
        input_embeds
    )
    out = _gather_kernel(indices.astype(jnp.int32), table_padded)
    return out[:, :, :DIM]


# speedup vs baseline: 1.1640x; 1.1640x over previous
import functools

import jax
import jax.numpy as jnp
from jax import lax
from jax.experimental import pallas as pl
from jax.experimental.pallas import tpu as pltpu
from jax.experimental.pallas import tpu_sc as plsc

VOCAB = 100000
DIM = 64
PAD_DIM = 128
BATCH = 4096
SEQ = 50
NUM_WORKERS = 32
BATCH_PER_W = BATCH // NUM_WORKERS
GROUP = 4
NBUF = 4
NUM_GROUPS = BATCH_PER_W // GROUP


def _make_kernel():
    mesh = plsc.VectorSubcoreMesh(core_axis_name="c", subcore_axis_name="s")

    @functools.partial(
        pl.kernel,
        mesh=mesh,
        out_type=jax.ShapeDtypeStruct((BATCH, SEQ, PAD_DIM), jnp.float32),
        scratch_types=[
            pltpu.VMEM((BATCH_PER_W, SEQ), jnp.int32),
            pltpu.VMEM((NBUF, GROUP, SEQ, PAD_DIM), jnp.float32),
            pltpu.SemaphoreType.DMA,
            pltpu.SemaphoreType.DMA,
        ],
        compiler_params=pltpu.CompilerParams(use_tc_tiling_on_sc=True),
    )
    def k(idx_hbm, table_hbm, out_hbm, idx_v, rows_v, sem_g, sem_s):
        wid = lax.axis_index("s") * 2 + lax.axis_index("c")
        base = wid * BATCH_PER_W

        pltpu.sync_copy(idx_hbm.at[pl.ds(base, BATCH_PER_W)], idx_v)

        def gather_descs(g):
            return [
                pltpu.make_async_copy(
                    table_hbm.at[idx_v.at[g * GROUP + j]],
                    rows_v.at[g % NBUF, j],
                    sem_g,
                )
                for j in range(GROUP)
            ]

        def store_desc(g):
            return pltpu.make_async_copy(
                rows_v.at[g % NBUF],
                out_hbm.at[pl.ds(base + g * GROUP, GROUP)],
                sem_s,
            )

        for g0 in range(2):
            for d in gather_descs(g0):
                d.start()

        def group_wait(g):
            pltpu.make_async_copy(
                out_hbm.at[pl.ds(0, GROUP)],
                rows_v.at[g % NBUF],
                sem_g,
            ).wait()

        def body(g, carry):
            group_wait(g)
            store_desc(g).start()

            @pl.when(g >= 2)
            def _():
                store_desc(g - 2).wait()

            @pl.when(g < NUM_GROUPS - 2)
            def _():
                for d in gather_descs(g + 2):
                    d.start()

            return carry

        lax.fori_loop(0, NUM_GROUPS, body, 0)
        store_desc(NUM_GROUPS - 2).wait()
        store_desc(NUM_GROUPS - 1).wait()

    return k


_gather_kernel = _make_kernel()


def kernel(indices, input_embeds):
    table_padded = jnp.pad(input_embeds, ((0, 0), (0, PAD_DIM - DIM)))
    out = _gather_kernel(indices.astype(jnp.int32), table_padded)
    return out[:, :, :DIM]


# speedup vs baseline: 1.1674x; 1.0029x over previous
import functools

import jax
import jax.numpy as jnp
from jax import lax
from jax.experimental import pallas as pl
from jax.experimental.pallas import tpu as pltpu
from jax.experimental.pallas import tpu_sc as plsc

VOCAB = 100000
DIM = 64
PAD_DIM = 128
BATCH = 4096
SEQ = 50
NUM_WORKERS = 32
BATCH_PER_W = BATCH // NUM_WORKERS
GROUP = 2
NBUF = 8
NUM_GROUPS = BATCH_PER_W // GROUP


def _make_kernel():
    mesh = plsc.VectorSubcoreMesh(core_axis_name="c", subcore_axis_name="s")

    @functools.partial(
        pl.kernel,
        mesh=mesh,
        out_type=jax.ShapeDtypeStruct((BATCH, SEQ, PAD_DIM), jnp.float32),
        scratch_types=[
            pltpu.VMEM((BATCH_PER_W, SEQ), jnp.int32),
            pltpu.VMEM((NBUF, GROUP, SEQ, PAD_DIM), jnp.float32),
            pltpu.SemaphoreType.DMA,
            pltpu.SemaphoreType.DMA,
        ],
        compiler_params=pltpu.CompilerParams(use_tc_tiling_on_sc=True),
    )
    def k(idx_hbm, table_hbm, out_hbm, idx_v, rows_v, sem_g, sem_s):
        wid = lax.axis_index("s") * 2 + lax.axis_index("c")
        base = wid * BATCH_PER_W

        pltpu.sync_copy(idx_hbm.at[pl.ds(base, BATCH_PER_W)], idx_v)

        def gather_descs(g):
            return [
                pltpu.make_async_copy(
                    table_hbm.at[idx_v.at[g * GROUP + j]],
                    rows_v.at[g % NBUF, j],
                    sem_g,
                )
                for j in range(GROUP)
            ]

        def store_desc(g):
            return pltpu.make_async_copy(
                rows_v.at[g % NBUF],
                out_hbm.at[pl.ds(base + g * GROUP, GROUP)],
                sem_s,
            )

        for g0 in range(4):
            for d in gather_descs(g0):
                d.start()

        def group_wait(g):
            pltpu.make_async_copy(
                out_hbm.at[pl.ds(0, GROUP)],
                rows_v.at[g % NBUF],
                sem_g,
            ).wait()

        def body(g, carry):
            group_wait(g)
            store_desc(g).start()

            @pl.when(g >= 4)
            def _():
                store_desc(g - 4).wait()

            @pl.when(g < NUM_GROUPS - 4)
            def _():
                for d in gather_descs(g + 4):
                    d.start()

            return carry

        lax.fori_loop(0, NUM_GROUPS, body, 0)
        for gt in range(4):
            store_desc(NUM_GROUPS - 4 + gt).wait()

    return k


_gather_kernel = _make_kernel()


def kernel(indices, input_embeds):
    table_padded = jnp.pad(input_embeds, ((0, 0), (0, PAD_DIM - DIM)))
    out = _gather_kernel(indices.astype(jnp.int32), table_padded)
    return out[:, :, :DIM]
